# baseline (speedup 1.0000x reference)
import jax
import jax.numpy as jnp
from jax import lax
from jax.experimental import pallas as pl
from jax.experimental.pallas import tpu as pltpu

IOU_THRESHOLD = 0.5
SCORE_THRESHOLD = 0.05
MAX_DETECTIONS = 300
N_BOXES = 20000

ROWS = 160
COLS = 128
CHUNK = 8
NCHUNK = ROWS // CHUNK
NEG = -1e9
VALID_MIN = -1e8
BIG_I = 1 << 30


def _nms_body(y1_ref, x1_ref, y2_ref, x2_ref, sc_ref, out_ref, s_ref):
    out_ref[...] = jnp.zeros_like(out_ref)

    s0 = sc_ref[...]
    s_init = jnp.where(s0 > SCORE_THRESHOLD, s0, NEG)
    s_ref[...] = s_init

    lane = lax.broadcasted_iota(jnp.int32, (1, COLS), 1)
    sub8 = lax.broadcasted_iota(jnp.int32, (CHUNK, COLS), 0)
    lane8 = lax.broadcasted_iota(jnp.int32, (CHUNK, COLS), 1)
    lin8 = sub8 * COLS + lane8

    cmax = jnp.full((1, COLS), NEG, dtype=jnp.float32)
    for t in range(NCHUNK):
        mt = jnp.max(s_init[t * CHUNK:(t + 1) * CHUNK, :])
        cmax = jnp.where(lane == t, mt, cmax)

    zero_v = jnp.zeros((CHUNK, COLS), jnp.float32)

    def cond(carry):
        k, live, cmax, ky1, kx1, ky2, kx2 = carry
        return (k < MAX_DETECTIONS) & live

    def body(carry):
        k, live, cmax, ky1, kx1, ky2, kx2 = carry

        m = jnp.max(cmax)
        valid = m > VALID_MIN
        t = jnp.min(jnp.where(cmax == m, lane, BIG_I))
        t = jnp.where(valid, t, 0)
        r0 = t * CHUNK

        chunk = s_ref[pl.ds(r0, CHUNK), :]
        p = jnp.min(jnp.where(chunk == m, lin8, BIG_I))
        p = jnp.where(valid, p, 0)
        sub = p // COLS
        c = p - sub * COLS
        r = r0 + sub

        oh = (lane == c).astype(jnp.float32)
        by1 = jnp.sum(y1_ref[pl.ds(r, 1), :] * oh)
        bx1 = jnp.sum(x1_ref[pl.ds(r, 1), :] * oh)
        by2 = jnp.sum(y2_ref[pl.ds(r, 1), :] * oh)
        bx2 = jnp.sum(x2_ref[pl.ds(r, 1), :] * oh)

        yy1 = jnp.maximum(by1, ky1)
        xx1 = jnp.maximum(bx1, kx1)
        yy2 = jnp.minimum(by2, ky2)
        xx2 = jnp.minimum(bx2, kx2)
        inter = jnp.maximum(yy2 - yy1, 0.0) * jnp.maximum(xx2 - xx1, 0.0)
        area_b = (by2 - by1) * (bx2 - bx1)
        karea = (ky2 - ky1) * (kx2 - kx1)
        iou = inter / jnp.maximum(area_b + karea - inter, 1e-9)
        sup = jnp.any(iou > IOU_THRESHOLD)
        keep_now = valid & (~sup)

        chunk_new = jnp.where(valid & (lin8 == p), NEG, chunk)
        s_ref[pl.ds(r0, CHUNK), :] = chunk_new
        cm_new = jnp.max(chunk_new)
        cmax = jnp.where(valid & (lane == t), cm_new, cmax)

        ksub = k // COLS
        klane = k - ksub * COLS
        slot = (sub8 == ksub) & (lane8 == klane) & keep_now
        ky1 = jnp.where(slot, by1, ky1)
        kx1 = jnp.where(slot, bx1, kx1)
        ky2 = jnp.where(slot, by2, ky2)
        kx2 = jnp.where(slot, bx2, kx2)

        kf = jnp.float32(keep_now)
        row = (
            jnp.where(lane == 0, by1, 0.0)
            + jnp.where(lane == 1, bx1, 0.0)
            + jnp.where(lane == 2, by2, 0.0)
            + jnp.where(lane == 3, bx2, 0.0)
            + jnp.where(lane == 4, m, 0.0)
        ) * kf
        out_ref[pl.ds(k, 1), :] = row

        k = k + jnp.where(keep_now, 1, 0)
        return (k, valid, cmax, ky1, kx1, ky2, kx2)

    lax.while_loop(
        cond, body,
        (jnp.int32(0), jnp.bool_(True), cmax, zero_v, zero_v, zero_v, zero_v),
    )


def kernel(boxes, scores):
    pad = ROWS * COLS - N_BOXES
    y1 = jnp.pad(boxes[:, 0], (0, pad)).reshape(ROWS, COLS)
    x1 = jnp.pad(boxes[:, 1], (0, pad)).reshape(ROWS, COLS)
    y2 = jnp.pad(boxes[:, 2], (0, pad)).reshape(ROWS, COLS)
    x2 = jnp.pad(boxes[:, 3], (0, pad)).reshape(ROWS, COLS)
    s = jnp.pad(scores, (0, pad)).reshape(ROWS, COLS)

    out = pl.pallas_call(
        _nms_body,
        out_shape=jax.ShapeDtypeStruct((304, COLS), jnp.float32),
        scratch_shapes=[pltpu.VMEM((ROWS, COLS), jnp.float32)],
    )(y1, x1, y2, x2, s)
    return out[:MAX_DETECTIONS, :5]


# speedup vs baseline: 1.4677x; 1.4677x over previous
import jax
import jax.numpy as jnp
from jax import lax
from jax.experimental import pallas as pl
from jax.experimental.pallas import tpu as pltpu

IOU_THRESHOLD = 0.5
SCORE_THRESHOLD = 0.05
MAX_DETECTIONS = 300
N_BOXES = 20000

ROWS = 160
COLS = 128
SROWS = 256
NEG = -1e9
VALID_MIN = -1e8
BIG_I = 1 << 30


def _nms_body(y1_ref, x1_ref, y2_ref, x2_ref, sc_ref, out_ref, ss_ref, si_ref):
    out_ref[...] = jnp.zeros_like(out_ref)

    lane = lax.broadcasted_iota(jnp.int32, (1, COLS), 1)
    sub8 = lax.broadcasted_iota(jnp.int32, (8, COLS), 0)
    lane8 = lax.broadcasted_iota(jnp.int32, (8, COLS), 1)
    slot_id = sub8 * COLS + lane8

    s0 = sc_ref[...]
    s_keyed = jnp.where(s0 > SCORE_THRESHOLD, s0, NEG)
    rid = lax.broadcasted_iota(jnp.int32, (SROWS, COLS), 0)
    cid = lax.broadcasted_iota(jnp.int32, (SROWS, COLS), 1)
    lin = rid * COLS + cid
    sfull = jnp.concatenate(
        [s_keyed, jnp.full((SROWS - ROWS, COLS), NEG, jnp.float32)], axis=0)
    ifull = lin

    def roll_up(x, j):
        return jnp.concatenate([x[j:], x[:j]], axis=0)

    def roll_down(x, j):
        return jnp.concatenate([x[SROWS - j:], x[:SROWS - j]], axis=0)

    k = 2
    while k <= SROWS:
        j = k // 2
        while j >= 1:
            bitj = (rid & j) != 0
            bitk = (rid & k) != 0
            ps = jnp.where(bitj, roll_down(sfull, j), roll_up(sfull, j))
            pi = jnp.where(bitj, roll_down(ifull, j), roll_up(ifull, j))
            gt = (sfull > ps) | ((sfull == ps) & (ifull < pi))
            take_cur = jnp.logical_xor(jnp.logical_xor(bitk, bitj), gt)
            sfull = jnp.where(take_cur, sfull, ps)
            ifull = jnp.where(take_cur, ifull, pi)
            j //= 2
        k *= 2

    ss_ref[...] = sfull
    si_ref[...] = ifull

    heads0 = sfull[0:1, :]
    hidx0 = ifull[0:1, :]
    hptr0 = jnp.zeros((1, COLS), jnp.int32)
    zero_v = jnp.zeros((8, COLS), jnp.float32)

    def cond(carry):
        kk, live, heads, hidx, hptr, ky1, kx1, ky2, kx2 = carry
        return (kk < MAX_DETECTIONS) & live

    def body(carry):
        kk, live, heads, hidx, hptr, ky1, kx1, ky2, kx2 = carry

        m11 = jnp.max(heads, axis=1, keepdims=True)
        valid = m11[0, 0] > VALID_MIN
        valid_v = m11 > VALID_MIN

        tie = jnp.where(heads == m11, hidx, BIG_I)
        pm = jnp.min(tie, axis=1, keepdims=True)
        oh = tie == pm
        ohf = oh.astype(jnp.float32)

        pm_s = jnp.min(tie)
        r = jnp.where(valid, jnp.minimum(pm_s // COLS, ROWS - 1), 0)
        c_v = pm & (COLS - 1)
        oc = (lane == c_v).astype(jnp.float32)

        by1 = jnp.sum(y1_ref[pl.ds(r, 1), :] * oc, axis=1, keepdims=True)
        bx1 = jnp.sum(x1_ref[pl.ds(r, 1), :] * oc, axis=1, keepdims=True)
        by2 = jnp.sum(y2_ref[pl.ds(r, 1), :] * oc, axis=1, keepdims=True)
        bx2 = jnp.sum(x2_ref[pl.ds(r, 1), :] * oc, axis=1, keepdims=True)

        h_s = jnp.min(jnp.where(oh, hptr, BIG_I))
        hn = jnp.where(valid, jnp.minimum(h_s + 1, SROWS - 1), 0)
        srow = ss_ref[pl.ds(hn, 1), :]
        irow = si_ref[pl.ds(hn, 1), :]
        upd = oh & valid_v
        heads = jnp.where(upd, srow, heads)
        hidx = jnp.where(upd, irow, hidx)
        hptr = jnp.where(upd, hptr + 1, hptr)

        yy1 = jnp.maximum(by1, ky1)
        xx1 = jnp.maximum(bx1, kx1)
        yy2 = jnp.minimum(by2, ky2)
        xx2 = jnp.minimum(bx2, kx2)
        inter = jnp.maximum(yy2 - yy1, 0.0) * jnp.maximum(xx2 - xx1, 0.0)
        area_b = (by2 - by1) * (bx2 - bx1)
        karea = (ky2 - ky1) * (kx2 - kx1)
        iou = inter / jnp.maximum(area_b + karea - inter, 1e-9)
        keep_now = valid & jnp.logical_not(jnp.any(iou > IOU_THRESHOLD))

        slot = (slot_id == kk) & keep_now
        ky1 = jnp.where(slot, by1, ky1)
        kx1 = jnp.where(slot, bx1, kx1)
        ky2 = jnp.where(slot, by2, ky2)
        kx2 = jnp.where(slot, bx2, kx2)

        kf = jnp.where(keep_now, 1.0, 0.0)
        row = (
            jnp.where(lane == 0, by1, 0.0)
            + jnp.where(lane == 1, bx1, 0.0)
            + jnp.where(lane == 2, by2, 0.0)
            + jnp.where(lane == 3, bx2, 0.0)
            + jnp.where(lane == 4, m11, 0.0)
        ) * kf
        out_ref[pl.ds(kk, 1), :] = row

        kk = kk + jnp.where(keep_now, 1, 0)
        return (kk, valid, heads, hidx, hptr, ky1, kx1, ky2, kx2)

    lax.while_loop(
        cond, body,
        (jnp.int32(0), jnp.bool_(True), heads0, hidx0, hptr0,
         zero_v, zero_v, zero_v, zero_v),
    )


def kernel(boxes, scores):
    pad = ROWS * COLS - N_BOXES
    y1 = jnp.pad(boxes[:, 0], (0, pad)).reshape(ROWS, COLS)
    x1 = jnp.pad(boxes[:, 1], (0, pad)).reshape(ROWS, COLS)
    y2 = jnp.pad(boxes[:, 2], (0, pad)).reshape(ROWS, COLS)
    x2 = jnp.pad(boxes[:, 3], (0, pad)).reshape(ROWS, COLS)
    s = jnp.pad(scores, (0, pad)).reshape(ROWS, COLS)

    out = pl.pallas_call(
        _nms_body,
        out_shape=jax.ShapeDtypeStruct((304, COLS), jnp.float32),
        scratch_shapes=[
            pltpu.VMEM((SROWS, COLS), jnp.float32),
            pltpu.VMEM((SROWS, COLS), jnp.int32),
        ],
    )(y1, x1, y2, x2, s)
    return out[:MAX_DETECTIONS, :5]


# speedup vs baseline: 2.4942x; 1.6995x over previous
import jax
import jax.numpy as jnp
from jax import lax
from jax.experimental import pallas as pl
from jax.experimental.pallas import tpu as pltpu

IOU_THRESHOLD = 0.5
SCORE_THRESHOLD = 0.05
MAX_DETECTIONS = 300
N_BOXES = 20000

ROWS = 160
COLS = 128
SROWS = 256
NEG = -1e9
VALID_MIN = -1e8
BIG_F = 1e9


def _nms_body(y1_ref, x1_ref, y2_ref, x2_ref, sc_ref, out_ref,
              ss_ref, si_ref, hp_ref):
    out_ref[...] = jnp.zeros_like(out_ref)
    for c in range(COLS):
        hp_ref[0, c] = 0

    lane = lax.broadcasted_iota(jnp.int32, (1, COLS), 1)
    lanef = lane.astype(jnp.float32)
    sub8 = lax.broadcasted_iota(jnp.int32, (8, COLS), 0)
    lane8 = lax.broadcasted_iota(jnp.int32, (8, COLS), 1)
    slot_id = sub8 * COLS + lane8

    s0 = sc_ref[...]
    s_keyed = jnp.where(s0 > SCORE_THRESHOLD, s0, NEG)
    rid = lax.broadcasted_iota(jnp.int32, (SROWS, COLS), 0)
    cid = lax.broadcasted_iota(jnp.int32, (SROWS, COLS), 1)
    linf = (rid * COLS + cid).astype(jnp.float32)
    sfull = jnp.concatenate(
        [s_keyed, jnp.full((SROWS - ROWS, COLS), NEG, jnp.float32)], axis=0)
    ifull = linf

    def roll_up(x, j):
        return jnp.concatenate([x[j:], x[:j]], axis=0)

    def roll_down(x, j):
        return jnp.concatenate([x[SROWS - j:], x[:SROWS - j]], axis=0)

    k = 2
    while k <= SROWS:
        j = k // 2
        while j >= 1:
            bitj = (rid & j) != 0
            bitk = (rid & k) != 0
            ps = jnp.where(bitj, roll_down(sfull, j), roll_up(sfull, j))
            pi = jnp.where(bitj, roll_down(ifull, j), roll_up(ifull, j))
            gt = (sfull > ps) | ((sfull == ps) & (ifull < pi))
            take_cur = jnp.logical_xor(jnp.logical_xor(bitk, bitj), gt)
            sfull = jnp.where(take_cur, sfull, ps)
            ifull = jnp.where(take_cur, ifull, pi)
            j //= 2
        k *= 2

    ss_ref[...] = sfull
    si_ref[...] = ifull

    heads0 = sfull[0:1, :]
    hidx0 = ifull[0:1, :]
    zero_v = jnp.zeros((8, COLS), jnp.float32)
    zero_1 = jnp.zeros((1, 1), jnp.float32)

    def cond(carry):
        (kk, pending, hlive, heads, hidx,
         cv, cy1, cx1, cy2, cx2, cs, ky1, kx1, ky2, kx2) = carry
        return (kk < MAX_DETECTIONS) & (pending | hlive)

    def body(carry):
        (kk, pending, hlive, heads, hidx,
         cv, cy1, cx1, cy2, cx2, cs, ky1, kx1, ky2, kx2) = carry

        m11 = jnp.max(heads, axis=1, keepdims=True)
        valid = m11[0, 0] > VALID_MIN
        valid_v = m11 > VALID_MIN
        tie = jnp.where(heads == m11, hidx, BIG_F)
        pmf = jnp.min(tie, axis=1, keepdims=True)
        oh = tie == pmf

        pm_s = jnp.min(tie).astype(jnp.int32)
        pm_s = jnp.where(valid, pm_s, 0)
        c_s = pm_s & (COLS - 1)
        r_s = jnp.minimum(pm_s >> 7, ROWS - 1)

        h_s = hp_ref[0, c_s]
        hp_ref[0, c_s] = h_s + jnp.where(valid, 1, 0)
        hn = jnp.minimum(h_s + 1, SROWS - 1)
        srow = ss_ref[pl.ds(hn, 1), :]
        irow = si_ref[pl.ds(hn, 1), :]
        upd = oh & valid_v
        heads = jnp.where(upd, srow, heads)
        hidx = jnp.where(upd, irow, hidx)

        c_v = pmf - jnp.floor(pmf * (1.0 / COLS)) * COLS
        oc = (lanef == c_v).astype(jnp.float32)
        ny1 = jnp.sum(y1_ref[pl.ds(r_s, 1), :] * oc, axis=1, keepdims=True)
        nx1 = jnp.sum(x1_ref[pl.ds(r_s, 1), :] * oc, axis=1, keepdims=True)
        ny2 = jnp.sum(y2_ref[pl.ds(r_s, 1), :] * oc, axis=1, keepdims=True)
        nx2 = jnp.sum(x2_ref[pl.ds(r_s, 1), :] * oc, axis=1, keepdims=True)

        yy1 = jnp.maximum(cy1, ky1)
        xx1 = jnp.maximum(cx1, kx1)
        yy2 = jnp.minimum(cy2, ky2)
        xx2 = jnp.minimum(cx2, kx2)
        inter = jnp.maximum(yy2 - yy1, 0.0) * jnp.maximum(xx2 - xx1, 0.0)
        area_b = (cy2 - cy1) * (cx2 - cx1)
        karea = (ky2 - ky1) * (kx2 - kx1)
        iou = inter / jnp.maximum(area_b + karea - inter, 1e-9)
        keep_now = pending & jnp.logical_not(jnp.any(iou > IOU_THRESHOLD))

        slot = (slot_id == kk) & keep_now
        ky1 = jnp.where(slot, cy1, ky1)
        kx1 = jnp.where(slot, cx1, kx1)
        ky2 = jnp.where(slot, cy2, ky2)
        kx2 = jnp.where(slot, cx2, kx2)

        kf = jnp.where(keep_now, 1.0, 0.0)
        row = (
            jnp.where(lane == 0, cy1, 0.0)
            + jnp.where(lane == 1, cx1, 0.0)
            + jnp.where(lane == 2, cy2, 0.0)
            + jnp.where(lane == 3, cx2, 0.0)
            + jnp.where(lane == 4, cs, 0.0)
        ) * kf
        out_ref[pl.ds(kk, 1), :] = row
        kk = kk + jnp.where(keep_now, 1, 0)

        return (kk, valid, valid, heads, hidx,
                valid, ny1, nx1, ny2, nx2, m11, ky1, kx1, ky2, kx2)

    lax.while_loop(
        cond, body,
        (jnp.int32(0), jnp.bool_(False), jnp.bool_(True), heads0, hidx0,
         jnp.bool_(False), zero_1, zero_1, zero_1, zero_1, zero_1,
         zero_v, zero_v, zero_v, zero_v),
    )


def kernel(boxes, scores):
    pad = ROWS * COLS - N_BOXES
    y1 = jnp.pad(boxes[:, 0], (0, pad)).reshape(ROWS, COLS)
    x1 = jnp.pad(boxes[:, 1], (0, pad)).reshape(ROWS, COLS)
    y2 = jnp.pad(boxes[:, 2], (0, pad)).reshape(ROWS, COLS)
    x2 = jnp.pad(boxes[:, 3], (0, pad)).reshape(ROWS, COLS)
    s = jnp.pad(scores, (0, pad)).reshape(ROWS, COLS)

    out = pl.pallas_call(
        _nms_body,
        out_shape=jax.ShapeDtypeStruct((304, COLS), jnp.float32),
        scratch_shapes=[
            pltpu.VMEM((SROWS, COLS), jnp.float32),
            pltpu.VMEM((SROWS, COLS), jnp.float32),
            pltpu.SMEM((1, COLS), jnp.int32),
        ],
    )(y1, x1, y2, x2, s)
    return out[:MAX_DETECTIONS, :5]


# speedup vs baseline: 3.0482x; 1.2221x over previous
import jax
import jax.numpy as jnp
from jax import lax
from jax.experimental import pallas as pl
from jax.experimental.pallas import tpu as pltpu

IOU_THRESHOLD = 0.5
SCORE_THRESHOLD = 0.05
MAX_DETECTIONS = 300
N_BOXES = 20000

ROWS = 160
COLS = 128
SROWS = 256
NEG = -1e9
VALID_MIN = -1e8
BIG_F = 1e9


def _nms_body(y1_ref, x1_ref, y2_ref, x2_ref, sc_ref, out_ref,
              ss_ref, si_ref, hp_ref):
    out_ref[...] = jnp.zeros_like(out_ref)
    for c in range(COLS):
        hp_ref[0, c] = 0

    lane = lax.broadcasted_iota(jnp.int32, (1, COLS), 1)
    lanef = lane.astype(jnp.float32)
    sub8 = lax.broadcasted_iota(jnp.int32, (8, COLS), 0)
    lane8 = lax.broadcasted_iota(jnp.int32, (8, COLS), 1)
    slot_id = sub8 * COLS + lane8

    s0 = sc_ref[...]
    s_keyed = jnp.where(s0 > SCORE_THRESHOLD, s0, NEG)
    rid = lax.broadcasted_iota(jnp.int32, (SROWS, COLS), 0)
    cid = lax.broadcasted_iota(jnp.int32, (SROWS, COLS), 1)
    linf = (rid * COLS + cid).astype(jnp.float32)
    sfull = jnp.concatenate(
        [s_keyed, jnp.full((SROWS - ROWS, COLS), NEG, jnp.float32)], axis=0)
    ifull = linf

    def roll_up(x, j):
        return jnp.concatenate([x[j:], x[:j]], axis=0)

    def roll_down(x, j):
        return jnp.concatenate([x[SROWS - j:], x[:SROWS - j]], axis=0)

    k = 2
    while k <= SROWS:
        j = k // 2
        while j >= 1:
            bitj = (rid & j) != 0
            bitk = (rid & k) != 0
            ps = jnp.where(bitj, roll_down(sfull, j), roll_up(sfull, j))
            pi = jnp.where(bitj, roll_down(ifull, j), roll_up(ifull, j))
            gt = (sfull > ps) | ((sfull == ps) & (ifull < pi))
            take_cur = jnp.logical_xor(jnp.logical_xor(bitk, bitj), gt)
            sfull = jnp.where(take_cur, sfull, ps)
            ifull = jnp.where(take_cur, ifull, pi)
            j //= 2
        k *= 2

    ss_ref[...] = sfull
    si_ref[...] = ifull

    heads0 = sfull[0:1, :]
    hidx0 = ifull[0:1, :]
    zero_v = jnp.zeros((8, COLS), jnp.float32)
    zero_1 = jnp.zeros((1, 1), jnp.float32)

    zero_r = jnp.zeros((1, COLS), jnp.float32)

    def cond(carry):
        (kk, pending, hlive, heads, hidx,
         r1, oc1, cs, ky1, kx1, ky2, kx2) = carry
        return (kk < MAX_DETECTIONS) & (pending | hlive)

    def body(carry):
        (kk, pending, hlive, heads, hidx,
         r1, oc1, cs, ky1, kx1, ky2, kx2) = carry

        cy1 = jnp.sum(y1_ref[pl.ds(r1, 1), :] * oc1, axis=1, keepdims=True)
        cx1 = jnp.sum(x1_ref[pl.ds(r1, 1), :] * oc1, axis=1, keepdims=True)
        cy2 = jnp.sum(y2_ref[pl.ds(r1, 1), :] * oc1, axis=1, keepdims=True)
        cx2 = jnp.sum(x2_ref[pl.ds(r1, 1), :] * oc1, axis=1, keepdims=True)

        m11 = jnp.max(heads, axis=1, keepdims=True)
        valid = m11[0, 0] > VALID_MIN
        valid_v = m11 > VALID_MIN
        tie = jnp.where(heads == m11, hidx, BIG_F)
        pmf = jnp.min(tie, axis=1, keepdims=True)
        oh = tie == pmf

        pm_s = jnp.min(tie).astype(jnp.int32)
        pm_s = jnp.where(valid, pm_s, 0)
        c_s = pm_s & (COLS - 1)
        r_s = jnp.minimum(pm_s >> 7, ROWS - 1)

        h_s = hp_ref[0, c_s]
        hp_ref[0, c_s] = h_s + jnp.where(valid, 1, 0)
        hn = jnp.minimum(h_s + 1, SROWS - 1)
        srow = ss_ref[pl.ds(hn, 1), :]
        irow = si_ref[pl.ds(hn, 1), :]
        upd = oh & valid_v
        heads = jnp.where(upd, srow, heads)
        hidx = jnp.where(upd, irow, hidx)

        c_v = pmf - jnp.floor(pmf * (1.0 / COLS)) * COLS
        oc = (lanef == c_v).astype(jnp.float32)

        yy1 = jnp.maximum(cy1, ky1)
        xx1 = jnp.maximum(cx1, kx1)
        yy2 = jnp.minimum(cy2, ky2)
        xx2 = jnp.minimum(cx2, kx2)
        inter = jnp.maximum(yy2 - yy1, 0.0) * jnp.maximum(xx2 - xx1, 0.0)
        area_b = (cy2 - cy1) * (cx2 - cx1)
        karea = (ky2 - ky1) * (kx2 - kx1)
        iou = inter / jnp.maximum(area_b + karea - inter, 1e-9)
        keep_now = pending & jnp.logical_not(jnp.any(iou > IOU_THRESHOLD))

        slot = (slot_id == kk) & keep_now
        ky1 = jnp.where(slot, cy1, ky1)
        kx1 = jnp.where(slot, cx1, kx1)
        ky2 = jnp.where(slot, cy2, ky2)
        kx2 = jnp.where(slot, cx2, kx2)

        kf = jnp.where(keep_now, 1.0, 0.0)
        row = (
            jnp.where(lane == 0, cy1, 0.0)
            + jnp.where(lane == 1, cx1, 0.0)
            + jnp.where(lane == 2, cy2, 0.0)
            + jnp.where(lane == 3, cx2, 0.0)
            + jnp.where(lane == 4, cs, 0.0)
        ) * kf
        out_ref[pl.ds(kk, 1), :] = row
        kk = kk + jnp.where(keep_now, 1, 0)

        return (kk, valid, valid, heads, hidx,
                r_s, oc, m11, ky1, kx1, ky2, kx2)

    lax.while_loop(
        cond, body,
        (jnp.int32(0), jnp.bool_(False), jnp.bool_(True), heads0, hidx0,
         jnp.int32(0), zero_r, zero_1,
         zero_v, zero_v, zero_v, zero_v),
    )


def kernel(boxes, scores):
    pad = ROWS * COLS - N_BOXES
    y1 = jnp.pad(boxes[:, 0], (0, pad)).reshape(ROWS, COLS)
    x1 = jnp.pad(boxes[:, 1], (0, pad)).reshape(ROWS, COLS)
    y2 = jnp.pad(boxes[:, 2], (0, pad)).reshape(ROWS, COLS)
    x2 = jnp.pad(boxes[:, 3], (0, pad)).reshape(ROWS, COLS)
    s = jnp.pad(scores, (0, pad)).reshape(ROWS, COLS)

    out = pl.pallas_call(
        _nms_body,
        out_shape=jax.ShapeDtypeStruct((304, COLS), jnp.float32),
        scratch_shapes=[
            pltpu.VMEM((SROWS, COLS), jnp.float32),
            pltpu.VMEM((SROWS, COLS), jnp.float32),
            pltpu.SMEM((1, COLS), jnp.int32),
        ],
    )(y1, x1, y2, x2, s)
    return out[:MAX_DETECTIONS, :5]


# speedup vs baseline: 3.2102x; 1.0531x over previous
import jax
import jax.numpy as jnp
from jax import lax
from jax.experimental import pallas as pl
from jax.experimental.pallas import tpu as pltpu

IOU_THRESHOLD = 0.5
SCORE_THRESHOLD = 0.05
MAX_DETECTIONS = 300
N_BOXES = 20000

ROWS = 160
COLS = 128
SROWS = 256
NEG = -1e9
VALID_MIN = -1e8
BIG_F = 1e9


def _nms_body(y1_ref, x1_ref, y2_ref, x2_ref, sc_ref, out_ref,
              ss_ref, si_ref):
    out_ref[...] = jnp.zeros_like(out_ref)

    lane = lax.broadcasted_iota(jnp.int32, (1, COLS), 1)
    lanef = lane.astype(jnp.float32)
    sub8 = lax.broadcasted_iota(jnp.int32, (8, COLS), 0)
    lane8 = lax.broadcasted_iota(jnp.int32, (8, COLS), 1)
    slot_id = sub8 * COLS + lane8

    s0 = sc_ref[...]
    s_keyed = jnp.where(s0 > SCORE_THRESHOLD, s0, NEG)
    rid = lax.broadcasted_iota(jnp.int32, (SROWS, COLS), 0)
    cid = lax.broadcasted_iota(jnp.int32, (SROWS, COLS), 1)
    linf = (rid * COLS + cid).astype(jnp.float32)
    sfull = jnp.concatenate(
        [s_keyed, jnp.full((SROWS - ROWS, COLS), NEG, jnp.float32)], axis=0)
    ifull = linf

    def roll_up(x, j):
        return jnp.concatenate([x[j:], x[:j]], axis=0)

    def roll_down(x, j):
        return jnp.concatenate([x[SROWS - j:], x[:SROWS - j]], axis=0)

    k = 2
    while k <= SROWS:
        j = k // 2
        while j >= 1:
            bitj = (rid & j) != 0
            bitk = (rid & k) != 0
            ps = jnp.where(bitj, roll_down(sfull, j), roll_up(sfull, j))
            pi = jnp.where(bitj, roll_down(ifull, j), roll_up(ifull, j))
            gt = (sfull > ps) | ((sfull == ps) & (ifull < pi))
            take_cur = jnp.logical_xor(jnp.logical_xor(bitk, bitj), gt)
            sfull = jnp.where(take_cur, sfull, ps)
            ifull = jnp.where(take_cur, ifull, pi)
            j //= 2
        k *= 2

    kfull = ifull * 256.0 + rid.astype(jnp.float32)
    ss_ref[...] = sfull
    si_ref[...] = kfull

    heads0 = sfull[0:1, :]
    hidx0 = kfull[0:1, :]
    n1s0 = roll_up(sfull, 1)[0:1, :]
    n1k0 = roll_up(kfull, 1)[0:1, :]
    n2s0 = roll_up(sfull, 2)[0:1, :]
    n2k0 = roll_up(kfull, 2)[0:1, :]
    zero_v = jnp.zeros((8, COLS), jnp.float32)
    zero_1 = jnp.zeros((1, 1), jnp.float32)

    zero_r = jnp.zeros((1, COLS), jnp.float32)

    def cond(carry):
        (kk, v1, v2, hlive, heads, hidx, n1s, n1k, n2s, n2k,
         r1, rl1, oc1, cs1, cy1, cx1, cy2, cx2, rowp,
         ky1, kx1, ky2, kx2) = carry
        return (kk < MAX_DETECTIONS) & ((hlive + v1 + v2) > 0)

    def body(carry):
        (kk, v1, v2, hlive, heads, hidx, n1s, n1k, n2s, n2k,
         r1, rl1, oc1, cs1, cy1, cx1, cy2, cx2, rowp,
         ky1, kx1, ky2, kx2) = carry

        rs_new = ss_ref[pl.ds(rl1, 1), :]
        rk_new = si_ref[pl.ds(rl1, 1), :]
        refill = (oc1 != 0.0) & (v1 > 0)
        n2s = jnp.where(refill, rs_new, n2s)
        n2k = jnp.where(refill, rk_new, n2k)

        ncy1 = jnp.sum(y1_ref[pl.ds(r1, 1), :] * oc1, axis=1, keepdims=True)
        ncx1 = jnp.sum(x1_ref[pl.ds(r1, 1), :] * oc1, axis=1, keepdims=True)
        ncy2 = jnp.sum(y2_ref[pl.ds(r1, 1), :] * oc1, axis=1, keepdims=True)
        ncx2 = jnp.sum(x2_ref[pl.ds(r1, 1), :] * oc1, axis=1, keepdims=True)
        nrowp = (
            jnp.where(lane == 0, ncy1, 0.0)
            + jnp.where(lane == 1, ncx1, 0.0)
            + jnp.where(lane == 2, ncy2, 0.0)
            + jnp.where(lane == 3, ncx2, 0.0)
            + jnp.where(lane == 4, cs1, 0.0)
        )

        m11 = jnp.max(heads, axis=1, keepdims=True)
        valid = m11[0, 0] > VALID_MIN
        valid_v = m11 > VALID_MIN
        tie = jnp.where(heads == m11, hidx, BIG_F)
        pmf = jnp.min(tie, axis=1, keepdims=True)
        oh = tie == pmf

        pm_s = jnp.min(tie).astype(jnp.int32)
        pm_s = jnp.where(valid, pm_s, 0)
        idx_s = pm_s >> 8
        r_s = jnp.minimum(idx_s >> 7, ROWS - 1)
        rl_new = jnp.minimum((pm_s & 255) + 3, SROWS - 1)

        upd = oh & valid_v
        heads = jnp.where(upd, n1s, heads)
        hidx = jnp.where(upd, n1k, hidx)
        n1s = jnp.where(upd, n2s, n1s)
        n1k = jnp.where(upd, n2k, n1k)

        idxv = jnp.floor(pmf * (1.0 / 256.0))
        c_v = idxv - jnp.floor(idxv * (1.0 / COLS)) * COLS
        oc = (lanef == c_v).astype(jnp.float32)

        yy1 = jnp.maximum(cy1, ky1)
        xx1 = jnp.maximum(cx1, kx1)
        yy2 = jnp.minimum(cy2, ky2)
        xx2 = jnp.minimum(cx2, kx2)
        inter = jnp.maximum(yy2 - yy1, 0.0) * jnp.maximum(xx2 - xx1, 0.0)
        area_b = (cy2 - cy1) * (cx2 - cx1)
        karea = (ky2 - ky1) * (kx2 - kx1)
        iou = inter / jnp.maximum(area_b + karea - inter, 1e-9)
        keep_now = (v2 > 0) & jnp.logical_not(jnp.any(iou > IOU_THRESHOLD))

        slot = (slot_id == kk) & keep_now
        ky1 = jnp.where(slot, cy1, ky1)
        kx1 = jnp.where(slot, cx1, kx1)
        ky2 = jnp.where(slot, cy2, ky2)
        kx2 = jnp.where(slot, cx2, kx2)

        kf = jnp.where(keep_now, 1.0, 0.0)
        row = rowp * kf
        out_ref[pl.ds(kk, 1), :] = row
        kk = kk + jnp.where(keep_now, 1, 0)

        validi = jnp.where(valid, jnp.int32(1), jnp.int32(0))
        v2n = jnp.where(v1 > 0, jnp.int32(1), jnp.int32(0))
        return (kk, validi, v2n, validi, heads, hidx, n1s, n1k, n2s, n2k,
                r_s, rl_new, oc, m11, ncy1, ncx1, ncy2, ncx2, nrowp,
                ky1, kx1, ky2, kx2)

    lax.while_loop(
        cond, body,
        (jnp.int32(0), jnp.int32(0), jnp.int32(0), jnp.int32(1),
         heads0, hidx0, n1s0, n1k0, n2s0, n2k0,
         jnp.int32(0), jnp.int32(3), zero_r, zero_1,
         zero_1, zero_1, zero_1, zero_1, zero_r,
         zero_v, zero_v, zero_v, zero_v),
    )


def kernel(boxes, scores):
    pad = ROWS * COLS - N_BOXES
    y1 = jnp.pad(boxes[:, 0], (0, pad)).reshape(ROWS, COLS)
    x1 = jnp.pad(boxes[:, 1], (0, pad)).reshape(ROWS, COLS)
    y2 = jnp.pad(boxes[:, 2], (0, pad)).reshape(ROWS, COLS)
    x2 = jnp.pad(boxes[:, 3], (0, pad)).reshape(ROWS, COLS)
    s = jnp.pad(scores, (0, pad)).reshape(ROWS, COLS)

    out = pl.pallas_call(
        _nms_body,
        out_shape=jax.ShapeDtypeStruct((304, COLS), jnp.float32),
        scratch_shapes=[
            pltpu.VMEM((SROWS, COLS), jnp.float32),
            pltpu.VMEM((SROWS, COLS), jnp.float32),
        ],
    )(y1, x1, y2, x2, s)
    return out[:MAX_DETECTIONS, :5]
